# baseline (speedup 1.0000x reference)
import functools

import jax
import jax.numpy as jnp
from jax import lax
from jax.experimental import pallas as pl
from jax.experimental.pallas import tpu as pltpu
from jax.experimental.pallas import tpu_sc as plsc

_N = 10000
_E = 320000
_NC = 2
_NS = 16
_NW = _NC * _NS
_CH = 40
_CHC = 80
_RQ = 624
_RT = _N - _NS * _RQ

_F32 = jnp.float32
_BF16 = jnp.bfloat16


def _pe_pack_perm():
    import numpy as _np
    pos = _np.empty(128, _np.int64)
    for b in range(4):
        for k in range(16):
            pos[b * 32 + 2 * k] = b * 32 + k
            pos[b * 32 + 2 * k + 1] = b * 32 + 16 + k
    return pos


def _sc_edge_pass(col_split: bool):
    mesh = plsc.VectorSubcoreMesh(core_axis_name="c", subcore_axis_name="s",
                                  num_cores=_NC, num_subcores=_NS)

    out_type = [jax.ShapeDtypeStruct((_NC * _N, 128), _F32)]
    scratch = (
        [pltpu.VMEM((_CH,), jnp.int32)] * 10
        + [pltpu.VMEM((_CH, 128), _F32)] * 4
        + [pltpu.VMEM((_CH, 64), jnp.int32)] * 2
        + [pltpu.VMEM((_CH, 128), _F32)] * 2
        + [pltpu.VMEM_SHARED((_N, 128), _F32)]
        + [pltpu.SemaphoreType.DMA] * 12
    )
    if col_split:
        edges_per_worker = _E // _NS
    else:
        edges_per_worker = _E // _NW
    n_chunks = edges_per_worker // _CH
    assert n_chunks % 2 == 0

    def body(a_hbm, b_hbm, pe_hbm, src_hbm, dst_hbm, z128_hbm, s_out, *rest):
        srcv, dstv, gidx, gsrc, sdst = (rest[0:2], rest[2:4], rest[4:6],
                                        rest[6:8], rest[8:10])
        av, bv, pev, hv = rest[10:12], rest[12:14], rest[14:16], rest[16:18]
        s_sh = rest[18]
        sema, semb, semp = rest[19:21], rest[21:23], rest[23:25]
        semsc, semis, semid = rest[25:27], rest[27:29], rest[29:31]

        cid = lax.axis_index("c")
        sid = lax.axis_index("s")

        row0 = sid * _RQ
        pltpu.sync_copy(z128_hbm.at[pl.ds(row0, _RQ)],
                        s_sh.at[pl.ds(row0, _RQ)])

        @pl.when(sid == _NS - 1)
        def _tail_zero():
            pltpu.sync_copy(z128_hbm.at[pl.ds(_NS * _RQ, _RT)],
                            s_sh.at[pl.ds(_NS * _RQ, _RT)])

        plsc.subcore_barrier()

        if col_split:
            ebase = sid * edges_per_worker
        else:
            ebase = (sid * _NC + cid) * edges_per_worker

        def idx_fire(jj, p):
            base = ebase + jj * _CH
            pltpu.async_copy(src_hbm.at[pl.ds(base, _CH)], srcv[p], semis[p])
            pltpu.async_copy(dst_hbm.at[pl.ds(base, _CH)], dstv[p], semid[p])

        def idx_wait(p):
            pltpu.make_async_copy(src_hbm.at[pl.ds(0, _CH)], srcv[p],
                                  semis[p]).wait()
            pltpu.make_async_copy(dst_hbm.at[pl.ds(0, _CH)], dstv[p],
                                  semid[p]).wait()

        def gathers_fire(jj, p):
            if col_split:
                for off in (0, 16, 24):
                    sl = pl.ds(off, 16)
                    gidx[p][sl] = dstv[p][sl] + cid * _N
                    gsrc[p][sl] = srcv[p][sl] + cid * _N
                ia, ib = gidx[p], gsrc[p]
                pe_base = cid * _E + ebase + jj * _CH
            else:
                ia, ib = dstv[p], srcv[p]
                pe_base = ebase + jj * _CH
            pltpu.async_copy(a_hbm.at[ia], av[p], sema[p])
            pltpu.async_copy(b_hbm.at[ib], bv[p], semb[p])
            pltpu.async_copy(pe_hbm.at[pl.ds(pe_base, _CH)], pev[p], semp[p])

        def gathers_wait(p):
            ia = gidx[p] if col_split else dstv[p]
            ib = gsrc[p] if col_split else srcv[p]
            pltpu.make_async_copy(a_hbm.at[ia], av[p], sema[p]).wait()
            pltpu.make_async_copy(b_hbm.at[ib], bv[p], semb[p]).wait()
            pltpu.make_async_copy(pe_hbm.at[pl.ds(0, _CH)], pev[p],
                                  semp[p]).wait()

        pltpu.sync_copy(src_hbm.at[pl.ds(ebase, _CH)], srcv[0])
        pltpu.sync_copy(dst_hbm.at[pl.ds(ebase, _CH)], dstv[0])
        gathers_fire(0, 0)
        idx_fire(1, 1)

        @pl.loop(0, n_chunks, step=2)
        def _macro(j):
            for p in (0, 1):
                jj = j + p
                q = 1 - p
                gathers_wait(p)

                @pl.when(jj + 1 < n_chunks)
                def _fire_next():
                    idx_wait(q)
                    gathers_fire(jj + 1, q)

                @pl.when(jj >= 2)
                def _drain_scatter():
                    pltpu.make_async_copy(hv[p], s_sh.at[sdst[p]],
                                          semsc[p]).wait()

                for off in (0, 16, 24):
                    sl = pl.ds(off, 16)
                    sdst[p][sl] = dstv[p][sl]

                @pl.loop(0, _CH)
                def _compute(r):
                    for cc in range(4):
                        pw = pev[p][r, pl.ds(cc * 16, 16)]
                        p0 = plsc.bitcast(lax.shift_left(pw, 16), _F32)
                        p1 = plsc.bitcast(
                            jnp.bitwise_and(pw, jnp.int32(-65536)), _F32)
                        s0 = pl.ds(cc * 32, 16)
                        s1 = pl.ds(cc * 32 + 16, 16)
                        hv[p][r, s0] = jnp.maximum(
                            av[p][r, s0] + bv[p][r, s0] + p0, 0.0)
                        hv[p][r, s1] = jnp.maximum(
                            av[p][r, s1] + bv[p][r, s1] + p1, 0.0)

                @pl.when(jj + 2 < n_chunks)
                def _prefetch_idx():
                    idx_fire(jj + 2, p)

                pltpu.async_copy(hv[p], s_sh.at[sdst[p]], semsc[p], add=True)

        for p in (0, 1):
            pltpu.make_async_copy(hv[p], s_sh.at[sdst[p]], semsc[p]).wait()

        plsc.subcore_barrier()

        pltpu.sync_copy(s_sh.at[pl.ds(row0, _RQ)],
                        s_out.at[pl.ds(cid * _N + row0, _RQ)])

        @pl.when(sid == _NS - 1)
        def _tail_wb():
            pltpu.sync_copy(s_sh.at[pl.ds(_NS * _RQ, _RT)],
                            s_out.at[pl.ds(cid * _N + _NS * _RQ, _RT)])

    return pl.kernel(body, out_type=out_type, mesh=mesh,
                     scratch_types=scratch,
                     compiler_params=pltpu.CompilerParams(
                         needs_layout_passes=False))


_sc_pass1 = _sc_edge_pass(col_split=True)
_sc_pass23 = _sc_edge_pass(col_split=False)


def _sc_counts_build():
    mesh = plsc.VectorSubcoreMesh(core_axis_name="c", subcore_axis_name="s",
                                  num_cores=_NC, num_subcores=_NS)
    out_type = [jax.ShapeDtypeStruct((_NC * _N, 16), _F32)]
    scratch = [
        pltpu.VMEM((_CHC,), jnp.int32),
        pltpu.VMEM((_CHC, 16), _F32),
        pltpu.VMEM_SHARED((_N, 16), _F32),
    ]
    epw = _E // _NW
    n_chunks = epw // _CHC

    def body(dst_hbm, z16_hbm, ones_hbm, cnt_out, dstv, onesv, cnt_sh):
        cid = lax.axis_index("c")
        sid = lax.axis_index("s")
        row0 = sid * _RQ
        pltpu.sync_copy(ones_hbm, onesv)
        pltpu.sync_copy(z16_hbm.at[pl.ds(row0, _RQ)],
                        cnt_sh.at[pl.ds(row0, _RQ)])

        @pl.when(sid == _NS - 1)
        def _tail_zero():
            pltpu.sync_copy(z16_hbm.at[pl.ds(_NS * _RQ, _RT)],
                            cnt_sh.at[pl.ds(_NS * _RQ, _RT)])

        plsc.subcore_barrier()
        ebase = (sid * _NC + cid) * epw

        @pl.loop(0, n_chunks)
        def _chunk(j):
            pltpu.sync_copy(dst_hbm.at[pl.ds(ebase + j * _CHC, _CHC)], dstv)
            pltpu.sync_copy(onesv, cnt_sh.at[dstv], add=True)

        plsc.subcore_barrier()
        pltpu.sync_copy(cnt_sh.at[pl.ds(row0, _RQ)],
                        cnt_out.at[pl.ds(cid * _N + row0, _RQ)])

        @pl.when(sid == _NS - 1)
        def _tail_wb():
            pltpu.sync_copy(cnt_sh.at[pl.ds(_NS * _RQ, _RT)],
                            cnt_out.at[pl.ds(cid * _N + _NS * _RQ, _RT)])

    return pl.kernel(body, out_type=out_type, mesh=mesh,
                     scratch_types=scratch)


_sc_counts = _sc_counts_build()


def _proj_body(g_ref, w_ref, b_ref, o_ref):
    o_ref[0] = (jnp.dot(g_ref[...], w_ref[0],
                        preferred_element_type=_F32)
                + b_ref[0]).astype(o_ref.dtype)


def _proj(g, wstack, bstack, bn, out_dtype=_F32):
    k, d, _ = wstack.shape
    n = g.shape[0]
    bstack = bstack.reshape(k, 1, 128)
    return pl.pallas_call(
        _proj_body,
        grid=(k, n // bn),
        in_specs=[
            pl.BlockSpec((bn, d), lambda kk, i: (i, 0)),
            pl.BlockSpec((1, d, 128), lambda kk, i: (kk, 0, 0)),
            pl.BlockSpec((1, 1, 128), lambda kk, i: (kk, 0, 0)),
        ],
        out_specs=pl.BlockSpec((1, bn, 128), lambda kk, i: (kk, i, 0)),
        out_shape=jax.ShapeDtypeStruct((k, n, 128), out_dtype),
    )(g, wstack, bstack)


def _post_body(col_split, final, h, *refs):
    if final:
        (s_ref, cnt_ref, w2_ref, b2_ref, g_ref, bt_ref,
         x_ref, fw1, fb1, fw2, fb2, fw3, fb3, fw4, fb4, o_ref) = refs
    else:
        s_ref, cnt_ref, w2_ref, b2_ref, g_ref, bt_ref, o_ref = refs
    if col_split:
        s = jnp.concatenate([s_ref[0], s_ref[1]], axis=-1)
    else:
        s = s_ref[0] + s_ref[1]
    cnt = cnt_ref[0, :, 0:1] + cnt_ref[1, :, 0:1]
    u = jnp.dot(s, w2_ref[...], preferred_element_type=_F32)
    u = u + cnt * b2_ref[0][None, :]
    u = jnp.maximum(u, 0.0)
    mu = jnp.mean(u, axis=-1, keepdims=True)
    var = jnp.mean((u - mu) ** 2, axis=-1, keepdims=True)
    y = (u - mu) / jnp.sqrt(var + 1e-5) * g_ref[0][None, :] + bt_ref[0][None, :]
    if final:
        f = jnp.maximum(jnp.dot(x_ref[...], fw1[...],
                                preferred_element_type=_F32) + fb1[0], 0.0)
        f = jnp.maximum(jnp.dot(f, fw2[...],
                                preferred_element_type=_F32) + fb2[0], 0.0)
        f = jnp.maximum(jnp.dot(f, fw3[...],
                                preferred_element_type=_F32) + fb3[0], 0.0)
        f = jnp.dot(f, fw4[...], preferred_element_type=_F32) + fb4[0]
        y = (y + f) * 0.5
    o_ref[...] = y


def _post(s, cnt, w2t, b2, g, bt, bn, col_split, final=False, ff=None):
    h = 256 if col_split else 128
    n = s.shape[1]
    full = lambda shape: pl.BlockSpec(shape, lambda i: tuple(0 for _ in shape))
    in_specs = [
        pl.BlockSpec((2, bn, 128), lambda i: (0, i, 0)),
        pl.BlockSpec((2, bn, 16), lambda i: (0, i, 0)),
        full(w2t.shape),
        full((1, h)),
        full((1, h)),
        full((1, h)),
    ]
    args = [s, cnt, w2t, b2.reshape(1, h), g.reshape(1, h), bt.reshape(1, h)]
    if final:
        x, f1t, f1b, f2t, f2b, f3t, f3b, f4t, f4b = ff
        in_specs += [pl.BlockSpec((bn, 128), lambda i: (i, 0)),
                     full(f1t.shape), full((1, 256)),
                     full(f2t.shape), full((1, 256)),
                     full(f3t.shape), full((1, 128)),
                     full(f4t.shape), full((1, 128))]
        args += [x, f1t, f1b.reshape(1, 256), f2t, f2b.reshape(1, 256),
                 f3t, f3b.reshape(1, 128), f4t, f4b.reshape(1, 128)]
    return pl.pallas_call(
        functools.partial(_post_body, col_split, final, h),
        grid=(n // bn,),
        in_specs=in_specs,
        out_specs=pl.BlockSpec((bn, h), lambda i: (i, 0)),
        out_shape=jax.ShapeDtypeStruct((n, h), _F32),
    )(*args)


def kernel(x, edge_index, edge_attr,
           c1_w1, c1_b1, c1_w2, c1_b2, n1_g, n1_b,
           c2_w1, c2_b1, c2_w2, c2_b2, n2_g, n2_b,
           c3_w1, c3_b1, c3_w2, c3_b2, n3_g, n3_b,
           f1_w, f1_b, f2_w, f2_b, f3_w, f3_b, f4_w, f4_b):
    src = edge_index[0]
    dst = edge_index[1]

    def as_i32(t):
        return lax.bitcast_convert_type(
            t.reshape(*t.shape[:-1], 64, 2), jnp.int32)
    zeros128 = jnp.zeros((128,), _F32)
    z128 = jnp.zeros((_N, 128), _F32)
    z16 = jnp.zeros((_N, 16), _F32)
    ones16 = jnp.ones((_CHC, 16), _F32)
    cnt = _sc_counts(dst, z16, ones16)[0].reshape(2, _N, 16)

    perm = _pe_pack_perm()
    we_stack = jnp.stack([
        c1_w1[0:128, 256:272].T, c1_w1[128:256, 256:272].T,
        c2_w1[:, 512:528].T, c3_w1[:, 256:272].T,
    ])[:, :, perm]
    pe_all = _proj(edge_attr, we_stack,
                   jnp.zeros((4, 128), _F32), bn=8000,
                   out_dtype=_BF16)
    pe_all = as_i32(pe_all)
    pe1 = pe_all[0:2].reshape(2 * _E, 64)
    pe2 = pe_all[2]
    pe3 = pe_all[3]

    w1_stack = jnp.stack([
        c1_w1[0:128, 0:128].T, c1_w1[128:256, 0:128].T,
        c1_w1[0:128, 128:256].T, c1_w1[128:256, 128:256].T,
    ])
    b1_stack = jnp.stack([c1_b1[:128], c1_b1[128:], zeros128, zeros128])
    ab1 = _proj(x, w1_stack, b1_stack, bn=2000)
    a1 = ab1[0:2].reshape(2 * _N, 128)
    b1 = ab1[2:4].reshape(2 * _N, 128)

    s1 = _sc_pass1(a1, b1, pe1, src, dst, z128)[0]
    g1 = _post(s1.reshape(2, _N, 128), cnt, c1_w2.T, c1_b2, n1_g, n1_b,
               bn=2000, col_split=True)

    w2_stack = jnp.stack([c2_w1[:, 0:256].T, c2_w1[:, 256:512].T])
    b2_stack = jnp.stack([c2_b1, zeros128])
    ab2 = _proj(g1, w2_stack, b2_stack, bn=2000)
    s2 = _sc_pass23(ab2[0], ab2[1], pe2, src, dst, z128)[0]
    g2 = _post(s2.reshape(2, _N, 128), cnt, c2_w2.T, c2_b2, n2_g, n2_b,
               bn=2000, col_split=False)

    w3_stack = jnp.stack([c3_w1[:, 0:128].T, c3_w1[:, 128:256].T])
    b3_stack = jnp.stack([c3_b1, zeros128])
    ab3 = _proj(g2, w3_stack, b3_stack, bn=2000)
    s3 = _sc_pass23(ab3[0], ab3[1], pe3, src, dst, z128)[0]
    out = _post(s3.reshape(2, _N, 128), cnt, c3_w2.T, c3_b2, n3_g, n3_b,
                bn=2000, col_split=False, final=True,
                ff=(x, f1_w.T, f1_b, f2_w.T, f2_b,
                    f3_w.T, f3_b, f4_w.T, f4_b))
    return out


# speedup vs baseline: 1.4975x; 1.4975x over previous
import functools

import jax
import jax.numpy as jnp
from jax import lax
from jax.experimental import pallas as pl
from jax.experimental.pallas import tpu as pltpu
from jax.experimental.pallas import tpu_sc as plsc

_N = 10000
_E = 320000
_NC = 2
_NS = 16
_NW = _NC * _NS
_CH = 40
_CHC = 80
_RQ = 624
_RT = _N - _NS * _RQ

_F32 = jnp.float32
_BF16 = jnp.bfloat16


def _pe_pack_perm():
    import numpy as _np
    pos = _np.empty(128, _np.int64)
    for b in range(4):
        for k in range(16):
            pos[b * 32 + 2 * k] = b * 32 + k
            pos[b * 32 + 2 * k + 1] = b * 32 + 16 + k
    return pos


def _sc_edge_pass(col_split: bool):
    mesh = plsc.VectorSubcoreMesh(core_axis_name="c", subcore_axis_name="s",
                                  num_cores=_NC, num_subcores=_NS)

    out_type = [jax.ShapeDtypeStruct((_NC * _N, 128), _F32)]
    scratch = (
        [pltpu.VMEM((_CH,), jnp.int32)] * 10
        + [pltpu.VMEM((_CH, 128), _F32)] * 4
        + [pltpu.VMEM((_CH, 64), jnp.int32)] * 2
        + [pltpu.VMEM((_CH, 128), _F32)] * 2
        + [pltpu.VMEM_SHARED((_N, 128), _F32)]
        + [pltpu.SemaphoreType.DMA] * 12
    )
    if col_split:
        edges_per_worker = _E // _NS
    else:
        edges_per_worker = _E // _NW
    n_chunks = edges_per_worker // _CH
    assert n_chunks % 2 == 0

    def body(a_hbm, b_hbm, pe_hbm, src_hbm, dst_hbm, z128_hbm, s_out, *rest):
        srcv, dstv, gidx, gsrc, sdst = (rest[0:2], rest[2:4], rest[4:6],
                                        rest[6:8], rest[8:10])
        av, bv, pev, hv = rest[10:12], rest[12:14], rest[14:16], rest[16:18]
        s_sh = rest[18]
        sema, semb, semp = rest[19:21], rest[21:23], rest[23:25]
        semsc, semis, semid = rest[25:27], rest[27:29], rest[29:31]

        cid = lax.axis_index("c")
        sid = lax.axis_index("s")

        row0 = sid * _RQ
        pltpu.sync_copy(z128_hbm.at[pl.ds(row0, _RQ)],
                        s_sh.at[pl.ds(row0, _RQ)])

        @pl.when(sid == _NS - 1)
        def _tail_zero():
            pltpu.sync_copy(z128_hbm.at[pl.ds(_NS * _RQ, _RT)],
                            s_sh.at[pl.ds(_NS * _RQ, _RT)])

        plsc.subcore_barrier()

        if col_split:
            ebase = sid * edges_per_worker
        else:
            ebase = (sid * _NC + cid) * edges_per_worker

        def idx_fire(jj, p):
            base = ebase + jj * _CH
            pltpu.async_copy(src_hbm.at[pl.ds(base, _CH)], srcv[p], semis[p])
            pltpu.async_copy(dst_hbm.at[pl.ds(base, _CH)], dstv[p], semid[p])

        def idx_wait(p):
            pltpu.make_async_copy(src_hbm.at[pl.ds(0, _CH)], srcv[p],
                                  semis[p]).wait()
            pltpu.make_async_copy(dst_hbm.at[pl.ds(0, _CH)], dstv[p],
                                  semid[p]).wait()

        def gathers_fire(jj, p):
            if col_split:
                for off in (0, 16, 24):
                    sl = pl.ds(off, 16)
                    gidx[p][sl] = dstv[p][sl] + cid * _N
                    gsrc[p][sl] = srcv[p][sl] + cid * _N
                ia, ib = gidx[p], gsrc[p]
                pe_base = cid * _E + ebase + jj * _CH
            else:
                ia, ib = dstv[p], srcv[p]
                pe_base = ebase + jj * _CH
            pltpu.async_copy(a_hbm.at[ia], av[p], sema[p])
            pltpu.async_copy(b_hbm.at[ib], bv[p], semb[p])
            pltpu.async_copy(pe_hbm.at[pl.ds(pe_base, _CH)], pev[p], semp[p])

        def gathers_wait(p):
            ia = gidx[p] if col_split else dstv[p]
            ib = gsrc[p] if col_split else srcv[p]
            pltpu.make_async_copy(a_hbm.at[ia], av[p], sema[p]).wait()
            pltpu.make_async_copy(b_hbm.at[ib], bv[p], semb[p]).wait()
            pltpu.make_async_copy(pe_hbm.at[pl.ds(0, _CH)], pev[p],
                                  semp[p]).wait()

        pltpu.sync_copy(src_hbm.at[pl.ds(ebase, _CH)], srcv[0])
        pltpu.sync_copy(dst_hbm.at[pl.ds(ebase, _CH)], dstv[0])
        gathers_fire(0, 0)
        idx_fire(1, 1)

        @pl.loop(0, n_chunks, step=2)
        def _macro(j):
            for p in (0, 1):
                jj = j + p
                q = 1 - p
                gathers_wait(p)

                @pl.when(jj + 1 < n_chunks)
                def _fire_next():
                    idx_wait(q)
                    gathers_fire(jj + 1, q)

                @pl.when(jj >= 2)
                def _drain_scatter():
                    pltpu.make_async_copy(hv[p], s_sh.at[sdst[p]],
                                          semsc[p]).wait()

                for off in (0, 16, 24):
                    sl = pl.ds(off, 16)
                    sdst[p][sl] = dstv[p][sl]

                @pl.loop(0, _CH)
                def _compute(r):
                    for cc in range(4):
                        pw = pev[p][r, pl.ds(cc * 16, 16)]
                        p0 = plsc.bitcast(lax.shift_left(pw, 16), _F32)
                        p1 = plsc.bitcast(
                            jnp.bitwise_and(pw, jnp.int32(-65536)), _F32)
                        s0 = pl.ds(cc * 32, 16)
                        s1 = pl.ds(cc * 32 + 16, 16)
                        hv[p][r, s0] = jnp.maximum(
                            av[p][r, s0] + bv[p][r, s0] + p0, 0.0)
                        hv[p][r, s1] = jnp.maximum(
                            av[p][r, s1] + bv[p][r, s1] + p1, 0.0)

                @pl.when(jj + 2 < n_chunks)
                def _prefetch_idx():
                    idx_fire(jj + 2, p)

                pltpu.async_copy(hv[p], s_sh.at[sdst[p]], semsc[p], add=True)

        for p in (0, 1):
            pltpu.make_async_copy(hv[p], s_sh.at[sdst[p]], semsc[p]).wait()

        plsc.subcore_barrier()

        pltpu.sync_copy(s_sh.at[pl.ds(row0, _RQ)],
                        s_out.at[pl.ds(cid * _N + row0, _RQ)])

        @pl.when(sid == _NS - 1)
        def _tail_wb():
            pltpu.sync_copy(s_sh.at[pl.ds(_NS * _RQ, _RT)],
                            s_out.at[pl.ds(cid * _N + _NS * _RQ, _RT)])

    return pl.kernel(body, out_type=out_type, mesh=mesh,
                     scratch_types=scratch,
                     compiler_params=pltpu.CompilerParams(
                         needs_layout_passes=False))


_sc_pass1 = _sc_edge_pass(col_split=True)
_sc_pass23 = _sc_edge_pass(col_split=False)


def _sc_counts_build():
    mesh = plsc.VectorSubcoreMesh(core_axis_name="c", subcore_axis_name="s",
                                  num_cores=_NC, num_subcores=_NS)
    out_type = [jax.ShapeDtypeStruct((_NC * _N, 16), _F32)]
    scratch = [
        pltpu.VMEM((_CHC,), jnp.int32),
        pltpu.VMEM((_CHC, 16), _F32),
        pltpu.VMEM_SHARED((_N, 16), _F32),
    ]
    epw = _E // _NW
    n_chunks = epw // _CHC

    def body(dst_hbm, z16_hbm, ones_hbm, cnt_out, dstv, onesv, cnt_sh):
        cid = lax.axis_index("c")
        sid = lax.axis_index("s")
        row0 = sid * _RQ
        pltpu.sync_copy(ones_hbm, onesv)
        pltpu.sync_copy(z16_hbm.at[pl.ds(row0, _RQ)],
                        cnt_sh.at[pl.ds(row0, _RQ)])

        @pl.when(sid == _NS - 1)
        def _tail_zero():
            pltpu.sync_copy(z16_hbm.at[pl.ds(_NS * _RQ, _RT)],
                            cnt_sh.at[pl.ds(_NS * _RQ, _RT)])

        plsc.subcore_barrier()
        ebase = (sid * _NC + cid) * epw

        @pl.loop(0, n_chunks)
        def _chunk(j):
            pltpu.sync_copy(dst_hbm.at[pl.ds(ebase + j * _CHC, _CHC)], dstv)
            pltpu.sync_copy(onesv, cnt_sh.at[dstv], add=True)

        plsc.subcore_barrier()
        pltpu.sync_copy(cnt_sh.at[pl.ds(row0, _RQ)],
                        cnt_out.at[pl.ds(cid * _N + row0, _RQ)])

        @pl.when(sid == _NS - 1)
        def _tail_wb():
            pltpu.sync_copy(cnt_sh.at[pl.ds(_NS * _RQ, _RT)],
                            cnt_out.at[pl.ds(cid * _N + _NS * _RQ, _RT)])

    return pl.kernel(body, out_type=out_type, mesh=mesh,
                     scratch_types=scratch)


_sc_counts = _sc_counts_build()


def _proj_body(g_ref, w_ref, b_ref, o_ref):
    o_ref[0] = (jnp.dot(g_ref[...], w_ref[0],
                        preferred_element_type=_F32)
                + b_ref[0])


def _proj_body_pack(g_ref, w_ref, b_ref, o_ref):
    y = (jnp.dot(g_ref[...], w_ref[0], preferred_element_type=_F32)
         + b_ref[0])
    xb = lax.bitcast_convert_type(y.astype(_BF16).astype(_F32), jnp.uint32)
    xb = lax.shift_right_logical(xb, jnp.uint32(16))
    parts = []
    for cc in range(4):
        lo = xb[:, cc * 32:cc * 32 + 16]
        hi = xb[:, cc * 32 + 16:cc * 32 + 32]
        parts.append(jnp.bitwise_or(lo, lax.shift_left(hi, jnp.uint32(16))))
    o_ref[0] = lax.bitcast_convert_type(
        jnp.concatenate(parts, axis=1), jnp.int32)


def _proj(g, wstack, bstack, bn, pack=False):
    k, d, _ = wstack.shape
    n = g.shape[0]
    bstack = bstack.reshape(k, 1, 128)
    cols = 64 if pack else 128
    return pl.pallas_call(
        _proj_body_pack if pack else _proj_body,
        grid=(k, n // bn),
        in_specs=[
            pl.BlockSpec((bn, d), lambda kk, i: (i, 0)),
            pl.BlockSpec((1, d, 128), lambda kk, i: (kk, 0, 0)),
            pl.BlockSpec((1, 1, 128), lambda kk, i: (kk, 0, 0)),
        ],
        out_specs=pl.BlockSpec((1, bn, cols), lambda kk, i: (kk, i, 0)),
        out_shape=jax.ShapeDtypeStruct(
            (k, n, cols), jnp.int32 if pack else _F32),
    )(g, wstack, bstack)


def _post_body(col_split, final, h, *refs):
    if final:
        (s_ref, cnt_ref, w2_ref, b2_ref, g_ref, bt_ref,
         x_ref, fw1, fb1, fw2, fb2, fw3, fb3, fw4, fb4, o_ref) = refs
    else:
        s_ref, cnt_ref, w2_ref, b2_ref, g_ref, bt_ref, o_ref = refs
    if col_split:
        s = jnp.concatenate([s_ref[0], s_ref[1]], axis=-1)
    else:
        s = s_ref[0] + s_ref[1]
    cnt = cnt_ref[0, :, 0:1] + cnt_ref[1, :, 0:1]
    u = jnp.dot(s, w2_ref[...], preferred_element_type=_F32)
    u = u + cnt * b2_ref[0][None, :]
    u = jnp.maximum(u, 0.0)
    mu = jnp.mean(u, axis=-1, keepdims=True)
    var = jnp.mean((u - mu) ** 2, axis=-1, keepdims=True)
    y = (u - mu) / jnp.sqrt(var + 1e-5) * g_ref[0][None, :] + bt_ref[0][None, :]
    if final:
        f = jnp.maximum(jnp.dot(x_ref[...], fw1[...],
                                preferred_element_type=_F32) + fb1[0], 0.0)
        f = jnp.maximum(jnp.dot(f, fw2[...],
                                preferred_element_type=_F32) + fb2[0], 0.0)
        f = jnp.maximum(jnp.dot(f, fw3[...],
                                preferred_element_type=_F32) + fb3[0], 0.0)
        f = jnp.dot(f, fw4[...], preferred_element_type=_F32) + fb4[0]
        y = (y + f) * 0.5
    o_ref[...] = y


def _post(s, cnt, w2t, b2, g, bt, bn, col_split, final=False, ff=None):
    h = 256 if col_split else 128
    n = s.shape[1]
    full = lambda shape: pl.BlockSpec(shape, lambda i: tuple(0 for _ in shape))
    in_specs = [
        pl.BlockSpec((2, bn, 128), lambda i: (0, i, 0)),
        pl.BlockSpec((2, bn, 16), lambda i: (0, i, 0)),
        full(w2t.shape),
        full((1, h)),
        full((1, h)),
        full((1, h)),
    ]
    args = [s, cnt, w2t, b2.reshape(1, h), g.reshape(1, h), bt.reshape(1, h)]
    if final:
        x, f1t, f1b, f2t, f2b, f3t, f3b, f4t, f4b = ff
        in_specs += [pl.BlockSpec((bn, 128), lambda i: (i, 0)),
                     full(f1t.shape), full((1, 256)),
                     full(f2t.shape), full((1, 256)),
                     full(f3t.shape), full((1, 128)),
                     full(f4t.shape), full((1, 128))]
        args += [x, f1t, f1b.reshape(1, 256), f2t, f2b.reshape(1, 256),
                 f3t, f3b.reshape(1, 128), f4t, f4b.reshape(1, 128)]
    return pl.pallas_call(
        functools.partial(_post_body, col_split, final, h),
        grid=(n // bn,),
        in_specs=in_specs,
        out_specs=pl.BlockSpec((bn, h), lambda i: (i, 0)),
        out_shape=jax.ShapeDtypeStruct((n, h), _F32),
    )(*args)


def kernel(x, edge_index, edge_attr,
           c1_w1, c1_b1, c1_w2, c1_b2, n1_g, n1_b,
           c2_w1, c2_b1, c2_w2, c2_b2, n2_g, n2_b,
           c3_w1, c3_b1, c3_w2, c3_b2, n3_g, n3_b,
           f1_w, f1_b, f2_w, f2_b, f3_w, f3_b, f4_w, f4_b):
    src = edge_index[0]
    dst = edge_index[1]

    def as_i32(t):
        return lax.bitcast_convert_type(
            t.reshape(*t.shape[:-1], 64, 2), jnp.int32)
    zeros128 = jnp.zeros((128,), _F32)
    z128 = jnp.zeros((_N, 128), _F32)
    z16 = jnp.zeros((_N, 16), _F32)
    ones16 = jnp.ones((_CHC, 16), _F32)
    cnt = _sc_counts(dst, z16, ones16)[0].reshape(2, _N, 16)

    we_stack = jnp.stack([
        c1_w1[0:128, 256:272].T, c1_w1[128:256, 256:272].T,
        c2_w1[:, 512:528].T, c3_w1[:, 256:272].T,
    ])
    pe_all = _proj(edge_attr, we_stack,
                   jnp.zeros((4, 128), _F32), bn=8000,
                   pack=True)
    pe1 = pe_all[0:2].reshape(2 * _E, 64)
    pe2 = pe_all[2]
    pe3 = pe_all[3]

    w1_stack = jnp.stack([
        c1_w1[0:128, 0:128].T, c1_w1[128:256, 0:128].T,
        c1_w1[0:128, 128:256].T, c1_w1[128:256, 128:256].T,
    ])
    b1_stack = jnp.stack([c1_b1[:128], c1_b1[128:], zeros128, zeros128])
    ab1 = _proj(x, w1_stack, b1_stack, bn=2000)
    a1 = ab1[0:2].reshape(2 * _N, 128)
    b1 = ab1[2:4].reshape(2 * _N, 128)

    s1 = _sc_pass1(a1, b1, pe1, src, dst, z128)[0]
    g1 = _post(s1.reshape(2, _N, 128), cnt, c1_w2.T, c1_b2, n1_g, n1_b,
               bn=2000, col_split=True)

    w2_stack = jnp.stack([c2_w1[:, 0:256].T, c2_w1[:, 256:512].T])
    b2_stack = jnp.stack([c2_b1, zeros128])
    ab2 = _proj(g1, w2_stack, b2_stack, bn=2000)
    s2 = _sc_pass23(ab2[0], ab2[1], pe2, src, dst, z128)[0]
    g2 = _post(s2.reshape(2, _N, 128), cnt, c2_w2.T, c2_b2, n2_g, n2_b,
               bn=2000, col_split=False)

    w3_stack = jnp.stack([c3_w1[:, 0:128].T, c3_w1[:, 128:256].T])
    b3_stack = jnp.stack([c3_b1, zeros128])
    ab3 = _proj(g2, w3_stack, b3_stack, bn=2000)
    s3 = _sc_pass23(ab3[0], ab3[1], pe3, src, dst, z128)[0]
    out = _post(s3.reshape(2, _N, 128), cnt, c3_w2.T, c3_b2, n3_g, n3_b,
                bn=2000, col_split=False, final=True,
                ff=(x, f1_w.T, f1_b, f2_w.T, f2_b,
                    f3_w.T, f3_b, f4_w.T, f4_b))
    return out


# speedup vs baseline: 1.6524x; 1.1035x over previous
import functools

import jax
import jax.numpy as jnp
from jax import lax
from jax.experimental import pallas as pl
from jax.experimental.pallas import tpu as pltpu
from jax.experimental.pallas import tpu_sc as plsc

_N = 10000
_E = 320000
_NC = 2
_NS = 16
_NW = _NC * _NS
_CH = 40
_CHC = 80
_RQ = 624
_RT = _N - _NS * _RQ

_F32 = jnp.float32


def _sc_edge_pass(col_split: bool):
    mesh = plsc.VectorSubcoreMesh(core_axis_name="c", subcore_axis_name="s",
                                  num_cores=_NC, num_subcores=_NS)

    out_type = [jax.ShapeDtypeStruct((_NC * _N, 128), _F32)]
    scratch = (
        [pltpu.VMEM((_CH,), jnp.int32)] * 10
        + [pltpu.VMEM((_CH, 128), _F32)] * 8
        + [pltpu.VMEM_SHARED((_N, 128), _F32)]
        + [pltpu.SemaphoreType.DMA] * 12
    )
    if col_split:
        edges_per_worker = _E // _NS
    else:
        edges_per_worker = _E // _NW
    n_chunks = edges_per_worker // _CH
    assert n_chunks % 2 == 0

    def body(a_hbm, b_hbm, pe_hbm, src_hbm, dst_hbm, z128_hbm, s_out, *rest):
        srcv, dstv, gidx, gsrc, sdst = (rest[0:2], rest[2:4], rest[4:6],
                                        rest[6:8], rest[8:10])
        av, bv, pev, hv = rest[10:12], rest[12:14], rest[14:16], rest[16:18]
        s_sh = rest[18]
        sema, semb, semp = rest[19:21], rest[21:23], rest[23:25]
        semsc, semis, semid = rest[25:27], rest[27:29], rest[29:31]

        cid = lax.axis_index("c")
        sid = lax.axis_index("s")

        row0 = sid * _RQ
        pltpu.sync_copy(z128_hbm.at[pl.ds(row0, _RQ)],
                        s_sh.at[pl.ds(row0, _RQ)])

        @pl.when(sid == _NS - 1)
        def _tail_zero():
            pltpu.sync_copy(z128_hbm.at[pl.ds(_NS * _RQ, _RT)],
                            s_sh.at[pl.ds(_NS * _RQ, _RT)])

        plsc.subcore_barrier()

        if col_split:
            ebase = sid * edges_per_worker
        else:
            ebase = (sid * _NC + cid) * edges_per_worker

        def idx_fire(jj, p):
            base = ebase + jj * _CH
            pltpu.async_copy(src_hbm.at[pl.ds(base, _CH)], srcv[p], semis[p])
            pltpu.async_copy(dst_hbm.at[pl.ds(base, _CH)], dstv[p], semid[p])

        def idx_wait(p):
            pltpu.make_async_copy(src_hbm.at[pl.ds(0, _CH)], srcv[p],
                                  semis[p]).wait()
            pltpu.make_async_copy(dst_hbm.at[pl.ds(0, _CH)], dstv[p],
                                  semid[p]).wait()

        def gathers_fire(jj, p):
            if col_split:
                for off in (0, 16, 24):
                    sl = pl.ds(off, 16)
                    gidx[p][sl] = dstv[p][sl] + cid * _N
                    gsrc[p][sl] = srcv[p][sl] + cid * _N
                ia, ib = gidx[p], gsrc[p]
                pe_base = cid * _E + ebase + jj * _CH
            else:
                ia, ib = dstv[p], srcv[p]
                pe_base = ebase + jj * _CH
            pltpu.async_copy(a_hbm.at[ia], av[p], sema[p])
            pltpu.async_copy(b_hbm.at[ib], bv[p], semb[p])
            pltpu.async_copy(pe_hbm.at[pl.ds(pe_base, _CH)], pev[p], semp[p])

        def gathers_wait(p):
            ia = gidx[p] if col_split else dstv[p]
            ib = gsrc[p] if col_split else srcv[p]
            pltpu.make_async_copy(a_hbm.at[ia], av[p], sema[p]).wait()
            pltpu.make_async_copy(b_hbm.at[ib], bv[p], semb[p]).wait()
            pltpu.make_async_copy(pe_hbm.at[pl.ds(0, _CH)], pev[p],
                                  semp[p]).wait()

        pltpu.sync_copy(src_hbm.at[pl.ds(ebase, _CH)], srcv[0])
        pltpu.sync_copy(dst_hbm.at[pl.ds(ebase, _CH)], dstv[0])
        gathers_fire(0, 0)
        idx_fire(1, 1)

        @pl.loop(0, n_chunks, step=2)
        def _macro(j):
            for p in (0, 1):
                jj = j + p
                q = 1 - p
                gathers_wait(p)

                @pl.when(jj + 1 < n_chunks)
                def _fire_next():
                    idx_wait(q)
                    gathers_fire(jj + 1, q)

                @pl.when(jj >= 2)
                def _drain_scatter():
                    pltpu.make_async_copy(hv[p], s_sh.at[sdst[p]],
                                          semsc[p]).wait()

                for off in (0, 16, 24):
                    sl = pl.ds(off, 16)
                    sdst[p][sl] = dstv[p][sl]

                @pl.loop(0, _CH)
                def _compute(r):
                    for cc in range(8):
                        sl = pl.ds(cc * 16, 16)
                        hv[p][r, sl] = jnp.maximum(
                            av[p][r, sl] + bv[p][r, sl] + pev[p][r, sl], 0.0)

                @pl.when(jj + 2 < n_chunks)
                def _prefetch_idx():
                    idx_fire(jj + 2, p)

                pltpu.async_copy(hv[p], s_sh.at[sdst[p]], semsc[p], add=True)

        for p in (0, 1):
            pltpu.make_async_copy(hv[p], s_sh.at[sdst[p]], semsc[p]).wait()

        plsc.subcore_barrier()

        pltpu.sync_copy(s_sh.at[pl.ds(row0, _RQ)],
                        s_out.at[pl.ds(cid * _N + row0, _RQ)])

        @pl.when(sid == _NS - 1)
        def _tail_wb():
            pltpu.sync_copy(s_sh.at[pl.ds(_NS * _RQ, _RT)],
                            s_out.at[pl.ds(cid * _N + _NS * _RQ, _RT)])

    return pl.kernel(body, out_type=out_type, mesh=mesh,
                     scratch_types=scratch)


_sc_pass1 = _sc_edge_pass(col_split=True)
_sc_pass23 = _sc_edge_pass(col_split=False)


def _sc_counts_build():
    mesh = plsc.VectorSubcoreMesh(core_axis_name="c", subcore_axis_name="s",
                                  num_cores=_NC, num_subcores=_NS)
    out_type = [jax.ShapeDtypeStruct((_NC * _N, 16), _F32)]
    scratch = [
        pltpu.VMEM((_CHC,), jnp.int32),
        pltpu.VMEM((_CHC, 16), _F32),
        pltpu.VMEM_SHARED((_N, 16), _F32),
    ]
    epw = _E // _NW
    n_chunks = epw // _CHC

    def body(dst_hbm, z16_hbm, ones_hbm, cnt_out, dstv, onesv, cnt_sh):
        cid = lax.axis_index("c")
        sid = lax.axis_index("s")
        row0 = sid * _RQ
        pltpu.sync_copy(ones_hbm, onesv)
        pltpu.sync_copy(z16_hbm.at[pl.ds(row0, _RQ)],
                        cnt_sh.at[pl.ds(row0, _RQ)])

        @pl.when(sid == _NS - 1)
        def _tail_zero():
            pltpu.sync_copy(z16_hbm.at[pl.ds(_NS * _RQ, _RT)],
                            cnt_sh.at[pl.ds(_NS * _RQ, _RT)])

        plsc.subcore_barrier()
        ebase = (sid * _NC + cid) * epw

        @pl.loop(0, n_chunks)
        def _chunk(j):
            pltpu.sync_copy(dst_hbm.at[pl.ds(ebase + j * _CHC, _CHC)], dstv)
            pltpu.sync_copy(onesv, cnt_sh.at[dstv], add=True)

        plsc.subcore_barrier()
        pltpu.sync_copy(cnt_sh.at[pl.ds(row0, _RQ)],
                        cnt_out.at[pl.ds(cid * _N + row0, _RQ)])

        @pl.when(sid == _NS - 1)
        def _tail_wb():
            pltpu.sync_copy(cnt_sh.at[pl.ds(_NS * _RQ, _RT)],
                            cnt_out.at[pl.ds(cid * _N + _NS * _RQ, _RT)])

    return pl.kernel(body, out_type=out_type, mesh=mesh,
                     scratch_types=scratch)


_sc_counts = _sc_counts_build()


def _proj_body(g_ref, w_ref, b_ref, o_ref):
    o_ref[0] = (jnp.dot(g_ref[...], w_ref[0],
                        preferred_element_type=_F32)
                + b_ref[0])


def _proj(g, wstack, bstack, bn):
    k, d, _ = wstack.shape
    n = g.shape[0]
    bstack = bstack.reshape(k, 1, 128)
    return pl.pallas_call(
        _proj_body,
        grid=(k, n // bn),
        in_specs=[
            pl.BlockSpec((bn, d), lambda kk, i: (i, 0)),
            pl.BlockSpec((1, d, 128), lambda kk, i: (kk, 0, 0)),
            pl.BlockSpec((1, 1, 128), lambda kk, i: (kk, 0, 0)),
        ],
        out_specs=pl.BlockSpec((1, bn, 128), lambda kk, i: (kk, i, 0)),
        out_shape=jax.ShapeDtypeStruct((k, n, 128), _F32),
    )(g, wstack, bstack)


def _post_body(col_split, final, h, *refs):
    if final:
        (s_ref, cnt_ref, w2_ref, b2_ref, g_ref, bt_ref,
         x_ref, fw1, fb1, fw2, fb2, fw3, fb3, fw4, fb4, o_ref) = refs
    else:
        s_ref, cnt_ref, w2_ref, b2_ref, g_ref, bt_ref, o_ref = refs
    if col_split:
        s = jnp.concatenate([s_ref[0], s_ref[1]], axis=-1)
    else:
        s = s_ref[0] + s_ref[1]
    cnt = cnt_ref[0, :, 0:1] + cnt_ref[1, :, 0:1]
    u = jnp.dot(s, w2_ref[...], preferred_element_type=_F32)
    u = u + cnt * b2_ref[0][None, :]
    u = jnp.maximum(u, 0.0)
    mu = jnp.mean(u, axis=-1, keepdims=True)
    var = jnp.mean((u - mu) ** 2, axis=-1, keepdims=True)
    y = (u - mu) / jnp.sqrt(var + 1e-5) * g_ref[0][None, :] + bt_ref[0][None, :]
    if final:
        f = jnp.maximum(jnp.dot(x_ref[...], fw1[...],
                                preferred_element_type=_F32) + fb1[0], 0.0)
        f = jnp.maximum(jnp.dot(f, fw2[...],
                                preferred_element_type=_F32) + fb2[0], 0.0)
        f = jnp.maximum(jnp.dot(f, fw3[...],
                                preferred_element_type=_F32) + fb3[0], 0.0)
        f = jnp.dot(f, fw4[...], preferred_element_type=_F32) + fb4[0]
        y = (y + f) * 0.5
    o_ref[...] = y


def _post_proj_body(col_split, h, refs):
    (s_ref, cnt_ref, w2_ref, b2_ref, g_ref, bt_ref,
     wn_ref, bn_ref, o_ref) = refs
    if col_split:
        s = jnp.concatenate([s_ref[0], s_ref[1]], axis=-1)
    else:
        s = s_ref[0] + s_ref[1]
    cnt = cnt_ref[0, :, 0:1] + cnt_ref[1, :, 0:1]
    u = jnp.dot(s, w2_ref[...], preferred_element_type=_F32)
    u = u + cnt * b2_ref[0][None, :]
    u = jnp.maximum(u, 0.0)
    mu = jnp.mean(u, axis=-1, keepdims=True)
    var = jnp.mean((u - mu) ** 2, axis=-1, keepdims=True)
    y = (u - mu) / jnp.sqrt(var + 1e-5) * g_ref[0][None, :] + bt_ref[0][None, :]
    for k in range(2):
        o_ref[k] = (jnp.dot(y, wn_ref[k], preferred_element_type=_F32)
                    + bn_ref[k])


def _post_proj(s, cnt, w2t, b2, g, bt, wnext, bnext, bn, col_split):
    h = 256 if col_split else 128
    n = s.shape[1]
    full = lambda shape: pl.BlockSpec(shape, lambda i: tuple(0 for _ in shape))
    in_specs = [
        pl.BlockSpec((2, bn, 128), lambda i: (0, i, 0)),
        pl.BlockSpec((2, bn, 16), lambda i: (0, i, 0)),
        full(w2t.shape),
        full((1, h)),
        full((1, h)),
        full((1, h)),
        full(wnext.shape),
        full((2, 1, 128)),
    ]
    args = [s, cnt, w2t, b2.reshape(1, h), g.reshape(1, h), bt.reshape(1, h),
            wnext, bnext.reshape(2, 1, 128)]
    return pl.pallas_call(
        lambda *refs: _post_proj_body(col_split, h, refs),
        grid=(n // bn,),
        in_specs=in_specs,
        out_specs=pl.BlockSpec((2, bn, 128), lambda i: (0, i, 0)),
        out_shape=jax.ShapeDtypeStruct((2, n, 128), _F32),
    )(*args)


def _post(s, cnt, w2t, b2, g, bt, bn, col_split, final=False, ff=None):
    h = 256 if col_split else 128
    n = s.shape[1]
    full = lambda shape: pl.BlockSpec(shape, lambda i: tuple(0 for _ in shape))
    in_specs = [
        pl.BlockSpec((2, bn, 128), lambda i: (0, i, 0)),
        pl.BlockSpec((2, bn, 16), lambda i: (0, i, 0)),
        full(w2t.shape),
        full((1, h)),
        full((1, h)),
        full((1, h)),
    ]
    args = [s, cnt, w2t, b2.reshape(1, h), g.reshape(1, h), bt.reshape(1, h)]
    if final:
        x, f1t, f1b, f2t, f2b, f3t, f3b, f4t, f4b = ff
        in_specs += [pl.BlockSpec((bn, 128), lambda i: (i, 0)),
                     full(f1t.shape), full((1, 256)),
                     full(f2t.shape), full((1, 256)),
                     full(f3t.shape), full((1, 128)),
                     full(f4t.shape), full((1, 128))]
        args += [x, f1t, f1b.reshape(1, 256), f2t, f2b.reshape(1, 256),
                 f3t, f3b.reshape(1, 128), f4t, f4b.reshape(1, 128)]
    return pl.pallas_call(
        functools.partial(_post_body, col_split, final, h),
        grid=(n // bn,),
        in_specs=in_specs,
        out_specs=pl.BlockSpec((bn, h), lambda i: (i, 0)),
        out_shape=jax.ShapeDtypeStruct((n, h), _F32),
    )(*args)


def kernel(x, edge_index, edge_attr,
           c1_w1, c1_b1, c1_w2, c1_b2, n1_g, n1_b,
           c2_w1, c2_b1, c2_w2, c2_b2, n2_g, n2_b,
           c3_w1, c3_b1, c3_w2, c3_b2, n3_g, n3_b,
           f1_w, f1_b, f2_w, f2_b, f3_w, f3_b, f4_w, f4_b):
    src = edge_index[0]
    dst = edge_index[1]
    zeros128 = jnp.zeros((128,), _F32)
    z128 = jnp.zeros((_N, 128), _F32)
    z16 = jnp.zeros((_N, 16), _F32)
    ones16 = jnp.ones((_CHC, 16), _F32)
    cnt = _sc_counts(dst, z16, ones16)[0].reshape(2, _N, 16)

    we_stack = jnp.stack([
        c1_w1[0:128, 256:272].T, c1_w1[128:256, 256:272].T,
        c2_w1[:, 512:528].T, c3_w1[:, 256:272].T,
    ])
    pe_all = _proj(edge_attr, we_stack,
                   jnp.zeros((4, 128), _F32), bn=8000)
    pe1 = pe_all[0:2].reshape(2 * _E, 128)
    pe2 = pe_all[2]
    pe3 = pe_all[3]

    w1_stack = jnp.stack([
        c1_w1[0:128, 0:128].T, c1_w1[128:256, 0:128].T,
        c1_w1[0:128, 128:256].T, c1_w1[128:256, 128:256].T,
    ])
    b1_stack = jnp.stack([c1_b1[:128], c1_b1[128:], zeros128, zeros128])
    ab1 = _proj(x, w1_stack, b1_stack, bn=2000)
    a1 = ab1[0:2].reshape(2 * _N, 128)
    b1 = ab1[2:4].reshape(2 * _N, 128)

    s1 = _sc_pass1(a1, b1, pe1, src, dst, z128)[0]

    w2_stack = jnp.stack([c2_w1[:, 0:256].T, c2_w1[:, 256:512].T])
    b2_stack = jnp.stack([c2_b1, zeros128])
    ab2 = _post_proj(s1.reshape(2, _N, 128), cnt, c1_w2.T, c1_b2, n1_g, n1_b,
                     w2_stack, b2_stack, bn=2000, col_split=True)
    s2 = _sc_pass23(ab2[0], ab2[1], pe2, src, dst, z128)[0]

    w3_stack = jnp.stack([c3_w1[:, 0:128].T, c3_w1[:, 128:256].T])
    b3_stack = jnp.stack([c3_b1, zeros128])
    ab3 = _post_proj(s2.reshape(2, _N, 128), cnt, c2_w2.T, c2_b2, n2_g, n2_b,
                     w3_stack, b3_stack, bn=2000, col_split=False)
    s3 = _sc_pass23(ab3[0], ab3[1], pe3, src, dst, z128)[0]
    out = _post(s3.reshape(2, _N, 128), cnt, c3_w2.T, c3_b2, n3_g, n3_b,
                bn=2000, col_split=False, final=True,
                ff=(x, f1_w.T, f1_b, f2_w.T, f2_b,
                    f3_w.T, f3_b, f4_w.T, f4_b))
    return out
